# baseline (speedup 1.0000x reference)
import jax
import jax.numpy as jnp
from jax.experimental import pallas as pl
from jax.experimental.pallas import tpu as pltpu

_NUM_EMB = 1024
_NUM_HEADS = 4
_DH = 256
_D = _NUM_HEADS * _DH
_KP = _DH + 128
_COMMITMENT_COST = 0.25
_BN = 512
_LOG2E = 1.4426950408889634


def _vq_kernel(x_ref, emb_ref, q_ref, codes_ref, loss_ref,
               e1_ref, e2_ref, xa_ref):
    i = pl.program_id(0)

    @pl.when(i == 0)
    def _init():
        lane0 = jax.lax.broadcasted_iota(jnp.int32, (_BN, 128), 1) == 0
        xa_ref[:, _DH:] = jnp.where(lane0, 1.0, 0.0)
        for h in range(_NUM_HEADS):
            eh = emb_ref[h]
            ee = jnp.sum(eh * eh, axis=1, keepdims=True)
            e1_ref[h, :, :_DH] = (2.0 * _LOG2E) * eh
            e1_ref[h, :, _DH:] = jnp.broadcast_to((-_LOG2E) * ee,
                                                  (_NUM_EMB, 128))
            e2_ref[h, :, :_DH] = eh
            e2_ref[h, :, _DH:] = jnp.ones((_NUM_EMB, 128), jnp.float32)
        loss_ref[...] = jnp.zeros_like(loss_ref)

    x = x_ref[...]
    lanes = jax.lax.broadcasted_iota(jnp.int32, (_BN, _NUM_EMB), 1)
    loss_part = jnp.float32(0.0)
    for h in range(_NUM_HEADS):
        xh = x[:, h * _DH:(h + 1) * _DH]
        xa_ref[:, :_DH] = xh
        l2 = jax.lax.dot_general(xa_ref[...], e1_ref[h],
                                 (((1,), (1,)), ((), ())),
                                 preferred_element_type=jnp.float32)
        m = jnp.max(l2, axis=1, keepdims=True)
        code = jnp.min(jnp.where(l2 == m, lanes, _NUM_EMB),
                       axis=1, keepdims=True)
        codes_ref[:, h:h + 1] = code
        p = jnp.exp2(l2 - m)
        q2 = jax.lax.dot_general(p, e2_ref[h], (((1,), (0,)), ((), ())),
                                 preferred_element_type=jnp.float32)
        qh = q2[:, :_DH] / q2[:, _DH:_DH + 1]
        q_ref[:, h * _DH:(h + 1) * _DH] = qh
        dh = qh - xh
        loss_part += jnp.sum(dh * dh)

    loss_ref[...] += jnp.full(loss_ref.shape, loss_part, jnp.float32)


def kernel(inputs, emb):
    x = inputs[:, 0, :]
    n = x.shape[0]
    q, codes, loss_acc = pl.pallas_call(
        _vq_kernel,
        grid=(n // _BN,),
        in_specs=[
            pl.BlockSpec((_BN, _D), lambda i: (i, 0)),
            pl.BlockSpec((_NUM_HEADS, _NUM_EMB, _DH), lambda i: (0, 0, 0)),
        ],
        out_specs=[
            pl.BlockSpec((_BN, _D), lambda i: (i, 0)),
            pl.BlockSpec((_BN, _NUM_HEADS), lambda i: (i, 0)),
            pl.BlockSpec((1, 1, 128), lambda i: (0, 0, 0)),
        ],
        out_shape=[
            jax.ShapeDtypeStruct((n, _D), jnp.float32),
            jax.ShapeDtypeStruct((n, _NUM_HEADS), jnp.int32),
            jax.ShapeDtypeStruct((1, 1, 128), jnp.float32),
        ],
        scratch_shapes=[
            pltpu.VMEM((_NUM_HEADS, _NUM_EMB, _KP), jnp.float32),
            pltpu.VMEM((_NUM_HEADS, _NUM_EMB, _KP), jnp.float32),
            pltpu.VMEM((_BN, _KP), jnp.float32),
        ],
    )(x, emb)
    loss = loss_acc[0, 0, 0] * (_COMMITMENT_COST / (n * _D))
    return loss, q.reshape(inputs.shape), codes


# speedup vs baseline: 1.0617x; 1.0617x over previous
import jax
import jax.numpy as jnp
from jax.experimental import pallas as pl
from jax.experimental.pallas import tpu as pltpu

_NUM_EMB = 1024
_NUM_HEADS = 4
_DH = 256
_D = _NUM_HEADS * _DH
_KP = _DH + 128
_COMMITMENT_COST = 0.25
_BN = 512


def _vq_kernel(x_ref, emb_ref, q_ref, codes_ref, loss_ref, ee_ref, e2_ref):
    i = pl.program_id(0)

    @pl.when(i == 0)
    def _init():
        for h in range(_NUM_HEADS):
            eh = emb_ref[h]
            ee_ref[h:h + 1, :] = jnp.sum(eh * eh, axis=1)[None, :]
            e2_ref[h, :, :_DH] = eh
            e2_ref[h, :, _DH:] = jnp.ones((_NUM_EMB, 128), jnp.float32)
        loss_ref[...] = jnp.zeros_like(loss_ref)

    x = x_ref[...]
    lanes = jax.lax.broadcasted_iota(jnp.int32, (_BN, _NUM_EMB), 1)
    loss_part = jnp.float32(0.0)
    for h in range(_NUM_HEADS):
        xh = x[:, h * _DH:(h + 1) * _DH]
        ip2 = jax.lax.dot_general(xh + xh, emb_ref[h], (((1,), (1,)), ((), ())),
                                  preferred_element_type=jnp.float32)
        logits = ip2 - ee_ref[h:h + 1, :]
        m = jnp.max(logits, axis=1, keepdims=True)
        t = logits - m
        code = jnp.min(jnp.where(t == 0.0, lanes, _NUM_EMB),
                       axis=1, keepdims=True)
        codes_ref[:, h:h + 1] = code
        p = jnp.exp(t)
        q2 = jax.lax.dot_general(p, e2_ref[h], (((1,), (0,)), ((), ())),
                                 preferred_element_type=jnp.float32)
        qh = q2[:, :_DH] / q2[:, _DH:_DH + 1]
        q_ref[:, h * _DH:(h + 1) * _DH] = qh
        dh = qh - xh
        loss_part += jnp.sum(dh * dh)

    loss_ref[...] += jnp.full(loss_ref.shape, loss_part, jnp.float32)


def kernel(inputs, emb):
    x = inputs[:, 0, :]
    n = x.shape[0]
    q, codes, loss_acc = pl.pallas_call(
        _vq_kernel,
        grid=(n // _BN,),
        in_specs=[
            pl.BlockSpec((_BN, _D), lambda i: (i, 0)),
            pl.BlockSpec((_NUM_HEADS, _NUM_EMB, _DH), lambda i: (0, 0, 0)),
        ],
        out_specs=[
            pl.BlockSpec((_BN, _D), lambda i: (i, 0)),
            pl.BlockSpec((_BN, _NUM_HEADS), lambda i: (i, 0)),
            pl.BlockSpec((1, 1, 128), lambda i: (0, 0, 0)),
        ],
        out_shape=[
            jax.ShapeDtypeStruct((n, _D), jnp.float32),
            jax.ShapeDtypeStruct((n, _NUM_HEADS), jnp.int32),
            jax.ShapeDtypeStruct((1, 1, 128), jnp.float32),
        ],
        scratch_shapes=[
            pltpu.VMEM((_NUM_HEADS, _NUM_EMB), jnp.float32),
            pltpu.VMEM((_NUM_HEADS, _NUM_EMB, _KP), jnp.float32),
        ],
    )(x, emb)
    loss = loss_acc[0, 0, 0] * (_COMMITMENT_COST / (n * _D))
    return loss, q.reshape(inputs.shape), codes


# speedup vs baseline: 1.4251x; 1.3424x over previous
import jax
import jax.numpy as jnp
from jax.experimental import pallas as pl
from jax.experimental.pallas import tpu as pltpu

_NUM_EMB = 1024
_NUM_HEADS = 4
_DH = 256
_D = _NUM_HEADS * _DH
_KP = _DH + 128
_COMMITMENT_COST = 0.25
_BN = 512


def _vq_kernel(x_ref, emb_ref, q_ref, codes_ref, loss_ref, ee_ref, e2_ref):
    i = pl.program_id(0)

    @pl.when(i == 0)
    def _init():
        for h in range(_NUM_HEADS):
            eh = emb_ref[h]
            ee_ref[h:h + 1, :] = jnp.sum(eh * eh, axis=1)[None, :]
            e2_ref[h, :, :_DH] = eh
            e2_ref[h, :, _DH:] = jnp.ones((_NUM_EMB, 128), jnp.float32)
        loss_ref[...] = jnp.zeros_like(loss_ref)

    x = x_ref[:, 0, :]
    lanes = jax.lax.broadcasted_iota(jnp.int32, (_BN, _NUM_EMB), 1)
    loss_part = jnp.float32(0.0)
    for h in range(_NUM_HEADS):
        xh = x[:, h * _DH:(h + 1) * _DH]
        ip2 = jax.lax.dot_general(xh + xh, emb_ref[h], (((1,), (1,)), ((), ())),
                                  preferred_element_type=jnp.float32)
        logits = ip2 - ee_ref[h:h + 1, :]
        m = jnp.max(logits, axis=1, keepdims=True)
        t = logits - m
        code = jnp.min(jnp.where(t == 0.0, lanes, _NUM_EMB),
                       axis=1, keepdims=True)
        codes_ref[:, h:h + 1] = code
        p = jnp.exp(t)
        q2 = jax.lax.dot_general(p, e2_ref[h], (((1,), (0,)), ((), ())),
                                 preferred_element_type=jnp.float32)
        qh = q2[:, :_DH] / q2[:, _DH:_DH + 1]
        q_ref[:, 0, h * _DH:(h + 1) * _DH] = qh
        dh = qh - xh
        loss_part += jnp.sum(dh * dh)

    loss_ref[...] += jnp.full(loss_ref.shape, loss_part, jnp.float32)


def kernel(inputs, emb):
    n = inputs.shape[0]
    q, codes, loss_acc = pl.pallas_call(
        _vq_kernel,
        grid=(n // _BN,),
        in_specs=[
            pl.BlockSpec((_BN, 1, _D), lambda i: (i, 0, 0)),
            pl.BlockSpec((_NUM_HEADS, _NUM_EMB, _DH), lambda i: (0, 0, 0)),
        ],
        out_specs=[
            pl.BlockSpec((_BN, 1, _D), lambda i: (i, 0, 0)),
            pl.BlockSpec((_BN, _NUM_HEADS), lambda i: (i, 0)),
            pl.BlockSpec((1, 1, 128), lambda i: (0, 0, 0)),
        ],
        out_shape=[
            jax.ShapeDtypeStruct((n, 1, _D), jnp.float32),
            jax.ShapeDtypeStruct((n, _NUM_HEADS), jnp.int32),
            jax.ShapeDtypeStruct((1, 1, 128), jnp.float32),
        ],
        scratch_shapes=[
            pltpu.VMEM((_NUM_HEADS, _NUM_EMB), jnp.float32),
            pltpu.VMEM((_NUM_HEADS, _NUM_EMB, _KP), jnp.float32),
        ],
    )(inputs, emb)
    loss = loss_acc[0, 0, 0] * (_COMMITMENT_COST / (n * _D))
    return loss, q, codes


# speedup vs baseline: 1.4499x; 1.0174x over previous
import jax
import jax.numpy as jnp
from jax.experimental import pallas as pl
from jax.experimental.pallas import tpu as pltpu

_NUM_EMB = 1024
_NUM_HEADS = 4
_DH = 256
_D = _NUM_HEADS * _DH
_KP = _DH + 128
_COMMITMENT_COST = 0.25
_BN = 512


def _vq_kernel(x_ref, emb_ref, q_ref, codes_ref, loss_ref, ee_ref, e2_ref):
    i = pl.program_id(0)

    @pl.when(i == 0)
    def _init():
        for h in range(_NUM_HEADS):
            eh = emb_ref[h]
            ee_ref[h:h + 1, :] = jnp.sum(eh * eh, axis=1)[None, :]
            e2_ref[h, :, :_DH] = eh
            e2_ref[h, :, _DH:] = jnp.ones((_NUM_EMB, 128), jnp.float32)
        loss_ref[...] = jnp.zeros_like(loss_ref)

    x = x_ref[...]
    lanes = jax.lax.broadcasted_iota(jnp.int32, (_BN, _NUM_EMB), 1)
    loss_part = jnp.float32(0.0)
    code_cols = []
    for h in range(_NUM_HEADS):
        xh = x[:, h * _DH:(h + 1) * _DH]
        ip2 = jax.lax.dot_general(xh + xh, emb_ref[h], (((1,), (1,)), ((), ())),
                                  preferred_element_type=jnp.float32)
        logits = ip2 - ee_ref[h:h + 1, :]
        m = jnp.max(logits, axis=1, keepdims=True)
        t = logits - m
        code_cols.append(jnp.min(jnp.where(t == 0.0, lanes, _NUM_EMB),
                                 axis=1, keepdims=True))
        p = jnp.exp(t)
        q2 = jax.lax.dot_general(p, e2_ref[h], (((1,), (0,)), ((), ())),
                                 preferred_element_type=jnp.float32)
        qh = q2[:, :_DH] * (1.0 / q2[:, _DH:_DH + 1])
        q_ref[:, h * _DH:(h + 1) * _DH] = qh
        dh = qh - xh
        loss_part += jnp.sum(dh * dh)
    codes_ref[...] = jnp.concatenate(code_cols, axis=1)

    loss_ref[...] += jnp.full(loss_ref.shape, loss_part, jnp.float32)


def kernel(inputs, emb):
    n = inputs.shape[0]
    q, codes, loss_acc = pl.pallas_call(
        _vq_kernel,
        grid=(n // _BN,),
        in_specs=[
            pl.BlockSpec((_BN, None, _D), lambda i: (i, 0, 0)),
            pl.BlockSpec((_NUM_HEADS, _NUM_EMB, _DH), lambda i: (0, 0, 0)),
        ],
        out_specs=[
            pl.BlockSpec((_BN, None, _D), lambda i: (i, 0, 0)),
            pl.BlockSpec((_BN, _NUM_HEADS), lambda i: (i, 0)),
            pl.BlockSpec((1, 1, 128), lambda i: (0, 0, 0)),
        ],
        out_shape=[
            jax.ShapeDtypeStruct((n, 1, _D), jnp.float32),
            jax.ShapeDtypeStruct((n, _NUM_HEADS), jnp.int32),
            jax.ShapeDtypeStruct((1, 1, 128), jnp.float32),
        ],
        scratch_shapes=[
            pltpu.VMEM((_NUM_HEADS, _NUM_EMB), jnp.float32),
            pltpu.VMEM((_NUM_HEADS, _NUM_EMB, _KP), jnp.float32),
        ],
    )(inputs, emb)
    loss = loss_acc[0, 0, 0] * (_COMMITMENT_COST / (n * _D))
    return loss, q, codes


# speedup vs baseline: 1.4778x; 1.0192x over previous
import jax
import jax.numpy as jnp
from jax.experimental import pallas as pl
from jax.experimental.pallas import tpu as pltpu

_NUM_EMB = 1024
_NUM_HEADS = 4
_DH = 256
_D = _NUM_HEADS * _DH
_KP = _DH + 128
_COMMITMENT_COST = 0.25
_BN = 512


def _vq_kernel(x_ref, emb_ref, q_ref, codes_ref, loss_ref, ee_ref, e2_ref):
    i = pl.program_id(0)

    @pl.when(i == 0)
    def _init():
        for h in range(_NUM_HEADS):
            eh = emb_ref[h]
            ee_ref[h:h + 1, :] = jnp.sum(eh * eh, axis=1)[None, :]
            e2_ref[h, :, :_DH] = eh
            e2_ref[h, :, _DH:] = jnp.ones((_NUM_EMB, 128), jnp.float32)
        loss_ref[...] = jnp.zeros_like(loss_ref)

    x = x_ref[...]
    revlanes = jax.lax.broadcasted_iota(jnp.int32, (_BN, _NUM_EMB), 1) ^ (
        _NUM_EMB - 1)
    loss_part = jnp.float32(0.0)
    code_cols = []
    for h in range(_NUM_HEADS):
        xh = x[:, h * _DH:(h + 1) * _DH]
        ip2 = jax.lax.dot_general(xh + xh, emb_ref[h], (((1,), (1,)), ((), ())),
                                  preferred_element_type=jnp.float32)
        logits = ip2 - ee_ref[h:h + 1, :]
        m = jnp.max(logits, axis=1, keepdims=True)
        t = logits - m
        ti = jax.lax.bitcast_convert_type(t, jnp.int32) | revlanes
        code_cols.append((_NUM_EMB - 1) - jnp.max(ti, axis=1, keepdims=True))
        p = jnp.exp(t)
        q2 = jax.lax.dot_general(p, e2_ref[h], (((1,), (0,)), ((), ())),
                                 preferred_element_type=jnp.float32)
        qh = q2[:, :_DH] * (1.0 / q2[:, _DH:_DH + 1])
        q_ref[:, h * _DH:(h + 1) * _DH] = qh
        dh = qh - xh
        loss_part += jnp.sum(dh * dh)
    codes_ref[...] = jnp.concatenate(code_cols, axis=1)

    loss_ref[...] += jnp.full(loss_ref.shape, loss_part, jnp.float32)


def kernel(inputs, emb):
    n = inputs.shape[0]
    q, codes, loss_acc = pl.pallas_call(
        _vq_kernel,
        grid=(n // _BN,),
        in_specs=[
            pl.BlockSpec((_BN, None, _D), lambda i: (i, 0, 0)),
            pl.BlockSpec((_NUM_HEADS, _NUM_EMB, _DH), lambda i: (0, 0, 0)),
        ],
        out_specs=[
            pl.BlockSpec((_BN, None, _D), lambda i: (i, 0, 0)),
            pl.BlockSpec((_BN, _NUM_HEADS), lambda i: (i, 0)),
            pl.BlockSpec((1, 1, 128), lambda i: (0, 0, 0)),
        ],
        out_shape=[
            jax.ShapeDtypeStruct((n, 1, _D), jnp.float32),
            jax.ShapeDtypeStruct((n, _NUM_HEADS), jnp.int32),
            jax.ShapeDtypeStruct((1, 1, 128), jnp.float32),
        ],
        scratch_shapes=[
            pltpu.VMEM((_NUM_HEADS, _NUM_EMB), jnp.float32),
            pltpu.VMEM((_NUM_HEADS, _NUM_EMB, _KP), jnp.float32),
        ],
    )(inputs, emb)
    loss = loss_acc[0, 0, 0] * (_COMMITMENT_COST / (n * _D))
    return loss, q, codes


# speedup vs baseline: 1.6752x; 1.1336x over previous
import jax
import jax.numpy as jnp
from jax.experimental import pallas as pl
from jax.experimental.pallas import tpu as pltpu

_NUM_EMB = 1024
_NUM_HEADS = 4
_DH = 256
_D = _NUM_HEADS * _DH
_KP = _DH + 128
_COMMITMENT_COST = 0.25
_BN = 512
_LOG2E = 1.4426950408889634


def _vq_kernel(x_ref, emb_ref, q_ref, codes_ref, loss_ref, ee_ref, e1_ref,
               e2_ref):
    i = pl.program_id(0)

    @pl.when(i == 0)
    def _init():
        for h in range(_NUM_HEADS):
            eh = emb_ref[h]
            ee_ref[h:h + 1, :] = (_LOG2E * jnp.sum(eh * eh, axis=1))[None, :]
            e1_ref[h] = (2.0 * _LOG2E) * eh
            e2_ref[h, :, :_DH] = eh
            e2_ref[h, :, _DH:] = jnp.ones((_NUM_EMB, 128), jnp.float32)
        loss_ref[...] = jnp.zeros_like(loss_ref)

    x = x_ref[...]
    revlanes = jax.lax.broadcasted_iota(jnp.int32, (_BN, _NUM_EMB), 1) ^ (
        _NUM_EMB - 1)
    loss_part = jnp.float32(0.0)
    code_cols = []
    for h in range(_NUM_HEADS):
        xh = x[:, h * _DH:(h + 1) * _DH]
        ip2 = jax.lax.dot_general(xh, e1_ref[h], (((1,), (1,)), ((), ())),
                                  preferred_element_type=jnp.float32)
        logits = ip2 - ee_ref[h:h + 1, :]
        m = jnp.max(logits, axis=1, keepdims=True)
        t = logits - m
        ti = jax.lax.bitcast_convert_type(t, jnp.int32) | revlanes
        code_cols.append((_NUM_EMB - 1) - jnp.max(ti, axis=1, keepdims=True))
        p = jnp.exp2(t)
        q2 = jax.lax.dot_general(p, e2_ref[h], (((1,), (0,)), ((), ())),
                                 preferred_element_type=jnp.float32)
        qh = q2[:, :_DH] * (1.0 / q2[:, _DH:_DH + 1])
        q_ref[:, h * _DH:(h + 1) * _DH] = qh
        dh = qh - xh
        loss_part += jnp.sum(dh * dh)
    codes_ref[...] = jnp.concatenate(code_cols, axis=1)

    loss_ref[...] += jnp.full(loss_ref.shape, loss_part, jnp.float32)


def kernel(inputs, emb):
    n = inputs.shape[0]
    q, codes, loss_acc = pl.pallas_call(
        _vq_kernel,
        grid=(n // _BN,),
        in_specs=[
            pl.BlockSpec((_BN, None, _D), lambda i: (i, 0, 0)),
            pl.BlockSpec((_NUM_HEADS, _NUM_EMB, _DH), lambda i: (0, 0, 0)),
        ],
        out_specs=[
            pl.BlockSpec((_BN, None, _D), lambda i: (i, 0, 0)),
            pl.BlockSpec((_BN, _NUM_HEADS), lambda i: (i, 0)),
            pl.BlockSpec((1, 1, 128), lambda i: (0, 0, 0)),
        ],
        out_shape=[
            jax.ShapeDtypeStruct((n, 1, _D), jnp.float32),
            jax.ShapeDtypeStruct((n, _NUM_HEADS), jnp.int32),
            jax.ShapeDtypeStruct((1, 1, 128), jnp.float32),
        ],
        scratch_shapes=[
            pltpu.VMEM((_NUM_HEADS, _NUM_EMB), jnp.float32),
            pltpu.VMEM((_NUM_HEADS, _NUM_EMB, _DH), jnp.float32),
            pltpu.VMEM((_NUM_HEADS, _NUM_EMB, _KP), jnp.float32),
        ],
    )(inputs, emb)
    loss = loss_acc[0, 0, 0] * (_COMMITMENT_COST / (n * _D))
    return loss, q, codes
